# baseline (speedup 1.0000x reference)
import functools
import math

import jax
import jax.numpy as jnp
import numpy as np
from jax import lax
from jax.experimental import pallas as pl
from jax.experimental.pallas import tpu as pltpu
from jax.experimental.pallas import tpu_sc as plsc

ANCHOR_AREAS = [1024.0, 4096.0, 16384.0, 65536.0, 262144.0]
ASPECT_RATIOS = [0.5, 1.0, 2.0]
SCALE_RATIOS = [1.0, 2.0 ** (1.0 / 3.0), 2.0 ** (2.0 / 3.0)]
INPUT_W, INPUT_H = 1024.0, 1024.0

NC, NS, L = 2, 16, 16
NW = NC * NS
A = 196416
A_PAD = 196608
GSIZE = 96
NG = A_PAD // GSIZE
NR = A // GSIZE
GPW = NG // NW
UNROLL = GSIZE // L
NBOX = 200
NBV = 13
TBOX = 256
REC = 8 + 6 * GSIZE
DEPTH = 8


def _anchor_channels():
    wh = []
    for s in ANCHOR_AREAS:
        for ar in ASPECT_RATIOS:
            h = math.sqrt(s / ar)
            w = ar * h
            for sr in SCALE_RATIOS:
                wh.append([w * sr, h * sr])
    num_fms = len(ANCHOR_AREAS)
    num_anchors = len(ASPECT_RATIOS) * len(SCALE_RATIOS)
    anchor_wh = np.asarray(wh, dtype=np.float32).reshape(num_fms, -1, 2)
    boxes = []
    for i in range(num_fms):
        fm_w = int(math.ceil(INPUT_W / 2.0 ** (i + 3)))
        fm_h = int(math.ceil(INPUT_H / 2.0 ** (i + 3)))
        gx = INPUT_W / fm_w
        gy = INPUT_H / fm_h
        xs = (np.arange(fm_w, dtype=np.float32) + 0.5) * gx
        ys = (np.arange(fm_h, dtype=np.float32) + 0.5) * gy
        xv, yv = np.meshgrid(xs, ys)
        xy = np.stack([xv, yv], axis=-1).reshape(fm_h, fm_w, 1, 2)
        xy = np.broadcast_to(xy, (fm_h, fm_w, num_anchors, 2))
        whb = np.broadcast_to(
            anchor_wh[i].reshape(1, 1, num_anchors, 2), (fm_h, fm_w, num_anchors, 2))
        box = np.concatenate([xy, whb], axis=3)
        boxes.append(box.reshape(-1, 4))
    anc = np.concatenate(boxes, axis=0).astype(np.float32)
    pad = np.zeros((A_PAD - A, 4), np.float32)
    pad[:, 2:] = 1.0
    anc = np.concatenate([anc, pad], axis=0)
    ax, ay, aw, ah = (anc[:, j].copy() for j in range(4))
    return (ax, ay, aw, ah,
            np.log(aw).astype(np.float32), np.log(ah).astype(np.float32))


def _build_records():
    ax, ay, aw, ah, law, lah = _anchor_channels()
    half = np.float32(0.5)
    altx = ax - aw * half
    arbx = ax + aw * half
    alty = ay - ah * half
    arby = ay + ah * half
    chan = np.stack([ax, ay, aw, ah, law, lah], 0).reshape(6, NG, GSIZE)
    gminx = altx.reshape(NG, GSIZE).min(1)
    gmaxx = arbx.reshape(NG, GSIZE).max(1)
    gminy = alty.reshape(NG, GSIZE).min(1)
    gmaxy = arby.reshape(NG, GSIZE).max(1)
    npad_g = (A_PAD - A) // GSIZE
    gminx[NG - npad_g:] = np.float32(3e38)
    gmaxx[NG - npad_g:] = np.float32(-3e38)
    rec = np.zeros((NG, REC), np.float32)
    rec[:, 0] = gminx
    rec[:, 1] = gmaxx
    rec[:, 2] = gminy
    rec[:, 3] = gmaxy
    rec[:, 8:] = np.transpose(chan, (1, 0, 2)).reshape(NG, 6 * GSIZE)
    rec = rec.reshape(GPW, NW, REC).transpose(1, 0, 2)
    return np.ascontiguousarray(rec).reshape(-1)


_REC = _build_records()


_LOGC = (1.44251703, -0.71789838, 0.45689417,
         -0.27736505, 0.12191414, -0.0260663)
_LN2 = 0.6931471805599453


def _vlog(x):
    i = plsc.bitcast(x, jnp.int32)
    e = (i >> 23) - 127
    m = plsc.bitcast((i & 0x007FFFFF) | 0x3F800000, jnp.float32)
    t = m - 1.0
    p = jnp.float32(_LOGC[5])
    for k in range(4, -1, -1):
        p = jnp.float32(_LOGC[k]) + t * p
    p = t * p
    return (e.astype(jnp.float32) + p) * jnp.float32(_LN2)

_MESH = plsc.VectorSubcoreMesh(
    core_axis_name="c", subcore_axis_name="s", num_cores=NC, num_subcores=NS)


@functools.partial(
    pl.kernel,
    out_type=[
        jax.ShapeDtypeStruct((A,), jnp.float32),
        jax.ShapeDtypeStruct((A,), jnp.float32),
        jax.ShapeDtypeStruct((A,), jnp.float32),
        jax.ShapeDtypeStruct((A,), jnp.float32),
        jax.ShapeDtypeStruct((A,), jnp.int32),
        jax.ShapeDtypeStruct((A,), jnp.float32),
    ],
    mesh=_MESH,
    compiler_params=pltpu.CompilerParams(needs_layout_passes=False),
    scratch_types=[
        pltpu.VMEM((GPW * REC,), jnp.float32),
        pltpu.VMEM((NBV * L * 4,), jnp.float32),
        pltpu.VMEM((12 * TBOX,), jnp.float32),
        pltpu.VMEM((TBOX,), jnp.int32),
        pltpu.VMEM((TBOX,), jnp.int32),
        pltpu.VMEM((GPW * GSIZE,), jnp.float32),
        pltpu.VMEM((GPW * GSIZE,), jnp.float32),
        pltpu.VMEM((GPW * GSIZE,), jnp.float32),
        pltpu.VMEM((GPW * GSIZE,), jnp.float32),
        pltpu.VMEM((GPW * GSIZE,), jnp.int32),
        pltpu.VMEM((GPW * GSIZE,), jnp.float32),
        pltpu.SemaphoreType.DMA,
    ],
)
def _sc_encode(rec_h, box_h, lab_h,
               lx_hbm, ly_hbm, lw_hbm, lh_hbm, cls_hbm, miou_hbm,
               rec_v, box_v, tab_v, lab_v, list_v,
               lx_v, ly_v, lw_v, lh_v, cls_v, miou_v, osem):
    w = lax.axis_index("s") * NC + lax.axis_index("c")
    pltpu.sync_copy(rec_h.at[pl.ds(w * (GPW * REC), GPW * REC)], rec_v)
    pltpu.sync_copy(box_h, box_v.at[pl.ds(0, NBOX * 4)])
    pltpu.sync_copy(lab_h, lab_v.at[pl.ds(0, NBOX)])

    zero = jnp.zeros((L,), jnp.float32)
    one = jnp.ones((L,), jnp.float32)
    zi = jnp.zeros((L,), jnp.int32)
    iota = lax.iota(jnp.int32, L)
    big = jnp.float32(3e38)

    for v in range(NBV):
        lane = iota + v * L
        idx = lane * 4
        x1 = plsc.load_gather(box_v, [idx])
        y1 = plsc.load_gather(box_v, [idx + 1])
        x2 = plsc.load_gather(box_v, [idx + 2])
        y2 = plsc.load_gather(box_v, [idx + 3])
        bx = (x1 + x2) * 0.5
        by = (y1 + y2) * 0.5
        bw = x2 - x1
        bh = y2 - y1
        hbw = bw * 0.5
        hbh = bh * 0.5
        valid = lane < NBOX
        sl = pl.ds(v * L, L)
        tab_v[sl] = jnp.where(valid, bx - hbw, big)
        tab_v[pl.ds(TBOX + v * L, L)] = jnp.where(valid, by - hbh, big)
        tab_v[pl.ds(2 * TBOX + v * L, L)] = jnp.where(valid, bx + hbw, -big)
        tab_v[pl.ds(3 * TBOX + v * L, L)] = jnp.where(valid, by + hbh, -big)
        tab_v[pl.ds(4 * TBOX + v * L, L)] = jnp.where(valid, bw * bh, zero)
        tab_v[pl.ds(5 * TBOX + v * L, L)] = bx
        tab_v[pl.ds(6 * TBOX + v * L, L)] = by
        tab_v[pl.ds(7 * TBOX + v * L, L)] = bw
        tab_v[pl.ds(8 * TBOX + v * L, L)] = bh
        tab_v[pl.ds(9 * TBOX + v * L, L)] = _vlog(jnp.where(valid, bw, one))
        tab_v[pl.ds(10 * TBOX + v * L, L)] = _vlog(jnp.where(valid, bh, one))

    def group_body(gl, carry):
        ro = gl * REC
        g = gl * NW + w
        rbase = lax.broadcast(ro, (L,))
        gminx = plsc.load_gather(rec_v, [rbase])
        gmaxx = plsc.load_gather(rec_v, [rbase + 1])
        gminy = plsc.load_gather(rec_v, [rbase + 2])
        gmaxy = plsc.load_gather(rec_v, [rbase + 3])

        off = zi
        for v in range(NBV):
            bltxv = tab_v[pl.ds(v * L, L)]
            bltyv = tab_v[pl.ds(TBOX + v * L, L)]
            brbxv = tab_v[pl.ds(2 * TBOX + v * L, L)]
            brbyv = tab_v[pl.ds(3 * TBOX + v * L, L)]
            incl = ((brbxv > gminx) & (bltxv < gmaxx)
                    & (brbyv > gminy) & (bltyv < gmaxy))
            pos = (off + plsc.cumsum(incl.astype(jnp.int32))) - 1
            plsc.store_scatter(list_v, [pos], iota + (v * L), mask=incl)
            off = off + plsc.all_reduce_population_count(incl)
        count = jnp.max(off)

        ab = ro + 8
        axs, ays, aws, ahs = [], [], [], []
        altx, alty, arbx, arby, area = [], [], [], [], []
        for k in range(UNROLL):
            axk = rec_v[pl.ds(ab + k * L, L)]
            ayk = rec_v[pl.ds(ab + GSIZE + k * L, L)]
            awk = rec_v[pl.ds(ab + 2 * GSIZE + k * L, L)]
            ahk = rec_v[pl.ds(ab + 3 * GSIZE + k * L, L)]
            hx = awk * 0.5
            hy = ahk * 0.5
            axs.append(axk)
            ays.append(ayk)
            aws.append(awk)
            ahs.append(ahk)
            altx.append(axk - hx)
            arbx.append(axk + hx)
            alty.append(ayk - hy)
            arby.append(ayk + hy)
            area.append(awk * ahk)

        def step(bc, cid):
            bi, bs, bid = bc
            bltx = plsc.load_gather(tab_v, [cid])
            blty = plsc.load_gather(tab_v, [cid + TBOX])
            brbx = plsc.load_gather(tab_v, [cid + 2 * TBOX])
            brby = plsc.load_gather(tab_v, [cid + 3 * TBOX])
            areab = plsc.load_gather(tab_v, [cid + 4 * TBOX])
            nbi, nbs, nbid = [], [], []
            for k in range(UNROLL):
                ltx = jnp.maximum(altx[k], bltx)
                lty = jnp.maximum(alty[k], blty)
                rbx = jnp.minimum(arbx[k], brbx)
                rby = jnp.minimum(arby[k], brby)
                wx = jnp.maximum(rbx - ltx, zero)
                wy = jnp.maximum(rby - lty, zero)
                inter = wx * wy
                sab = area[k] + areab
                p = inter * bs[k] > bi[k] * sab
                nbi.append(jnp.where(p, inter, bi[k]))
                nbs.append(jnp.where(p, sab, bs[k]))
                nbid.append(jnp.where(p, cid, bid[k]))
            return (tuple(nbi), tuple(nbs), tuple(nbid))

        def box_body(it, bc):
            cid = plsc.load_gather(list_v, [lax.broadcast(it, (L,))])
            return step(bc, cid)

        init = (tuple(zero for _ in range(UNROLL)),
                tuple(one for _ in range(UNROLL)),
                tuple(zi for _ in range(UNROLL)))
        binter, bsab, bid = lax.fori_loop(0, count, box_body, init)

        for k in range(UNROLL):
            lo = gl * GSIZE + k * L
            idk = bid[k]
            bxg = plsc.load_gather(tab_v, [idk + 5 * TBOX])
            byg = plsc.load_gather(tab_v, [idk + 6 * TBOX])
            lbw = plsc.load_gather(tab_v, [idk + 9 * TBOX])
            lbh = plsc.load_gather(tab_v, [idk + 10 * TBOX])
            areab = plsc.load_gather(tab_v, [idk + 4 * TBOX])
            lab = plsc.load_gather(lab_v, [idk])
            lx = (bxg - axs[k]) / aws[k]
            ly = (byg - ays[k]) / ahs[k]
            lw = lbw - rec_v[pl.ds(ab + 4 * GSIZE + k * L, L)]
            lh = lbh - rec_v[pl.ds(ab + 5 * GSIZE + k * L, L)]
            denom = (area[k] + areab) - binter[k]
            miou = binter[k] / denom
            cls = jnp.where(miou > 0.5, lab + 1,
                            jnp.where(miou > 0.4, -1, 0))
            sl = pl.ds(lo, L)
            lx_v[sl] = lx
            ly_v[sl] = ly
            lw_v[sl] = lw
            lh_v[sl] = lh
            cls_v[sl] = cls
            miou_v[sl] = miou

        @pl.when(g < NR)
        def _issue_out():
            src = pl.ds(gl * GSIZE, GSIZE)
            dst = pl.ds(g * GSIZE, GSIZE)
            pltpu.async_copy(lx_v.at[src], lx_hbm.at[dst], osem)
            pltpu.async_copy(ly_v.at[src], ly_hbm.at[dst], osem)
            pltpu.async_copy(lw_v.at[src], lw_hbm.at[dst], osem)
            pltpu.async_copy(lh_v.at[src], lh_hbm.at[dst], osem)
            pltpu.async_copy(cls_v.at[src], cls_hbm.at[dst], osem)
            pltpu.async_copy(miou_v.at[src], miou_hbm.at[dst], osem)

        @pl.when(gl >= DEPTH)
        def _drain_one():
            pltpu.make_async_copy(
                lx_hbm.at[pl.ds(0, 6 * GSIZE)],
                lx_v.at[pl.ds(0, 6 * GSIZE)], osem).wait()

        return carry

    lax.fori_loop(0, GPW, group_body, 0)

    @pl.when(w < NR - (GPW - 1) * NW)
    def _drain_full():
        pltpu.make_async_copy(
            lx_hbm.at[pl.ds(0, DEPTH * 6 * GSIZE)],
            lx_v.at[pl.ds(0, DEPTH * 6 * GSIZE)], osem).wait()

    @pl.when(w >= NR - (GPW - 1) * NW)
    def _drain_short():
        pltpu.make_async_copy(
            lx_hbm.at[pl.ds(0, (DEPTH - 1) * 6 * GSIZE)],
            lx_v.at[pl.ds(0, (DEPTH - 1) * 6 * GSIZE)], osem).wait()


def kernel(boxes, labels, input_size):
    del input_size
    boxes = boxes.astype(jnp.float32).reshape(-1)
    labels = labels.astype(jnp.int32)
    rec = jnp.asarray(_REC)
    lx, ly, lw, lh, cls, miou = _sc_encode(rec, boxes, labels)
    loc = jnp.stack([lx, ly, lw, lh], axis=1)
    return loc, cls, miou


# speedup vs baseline: 1.1197x; 1.1197x over previous
import functools
import math

import jax
import jax.numpy as jnp
import numpy as np
from jax import lax
from jax.experimental import pallas as pl
from jax.experimental.pallas import tpu as pltpu
from jax.experimental.pallas import tpu_sc as plsc

ANCHOR_AREAS = [1024.0, 4096.0, 16384.0, 65536.0, 262144.0]
ASPECT_RATIOS = [0.5, 1.0, 2.0]
SCALE_RATIOS = [1.0, 2.0 ** (1.0 / 3.0), 2.0 ** (2.0 / 3.0)]
INPUT_W, INPUT_H = 1024.0, 1024.0

NC, NS, L = 2, 16, 16
NW = NC * NS
A = 196416
A_PAD = 196608
GSIZE = 64
NG = A_PAD // GSIZE
NR = A // GSIZE
GPW = NG // NW
UNROLL = GSIZE // L
NBOX = 200
NBV = 13
TBOX = 256
REC = 392
DEPTH = 8


def _anchor_channels():
    wh = []
    for s in ANCHOR_AREAS:
        for ar in ASPECT_RATIOS:
            h = math.sqrt(s / ar)
            w = ar * h
            for sr in SCALE_RATIOS:
                wh.append([w * sr, h * sr])
    num_fms = len(ANCHOR_AREAS)
    num_anchors = len(ASPECT_RATIOS) * len(SCALE_RATIOS)
    anchor_wh = np.asarray(wh, dtype=np.float32).reshape(num_fms, -1, 2)
    boxes = []
    for i in range(num_fms):
        fm_w = int(math.ceil(INPUT_W / 2.0 ** (i + 3)))
        fm_h = int(math.ceil(INPUT_H / 2.0 ** (i + 3)))
        gx = INPUT_W / fm_w
        gy = INPUT_H / fm_h
        xs = (np.arange(fm_w, dtype=np.float32) + 0.5) * gx
        ys = (np.arange(fm_h, dtype=np.float32) + 0.5) * gy
        xv, yv = np.meshgrid(xs, ys)
        xy = np.stack([xv, yv], axis=-1).reshape(fm_h, fm_w, 1, 2)
        xy = np.broadcast_to(xy, (fm_h, fm_w, num_anchors, 2))
        whb = np.broadcast_to(
            anchor_wh[i].reshape(1, 1, num_anchors, 2), (fm_h, fm_w, num_anchors, 2))
        box = np.concatenate([xy, whb], axis=3)
        boxes.append(box.reshape(-1, 4))
    anc = np.concatenate(boxes, axis=0).astype(np.float32)
    pad = np.zeros((A_PAD - A, 4), np.float32)
    pad[:, 2:] = 1.0
    anc = np.concatenate([anc, pad], axis=0)
    ax, ay, aw, ah = (anc[:, j].copy() for j in range(4))
    return (ax, ay, aw, ah,
            np.log(aw).astype(np.float32), np.log(ah).astype(np.float32))


def _build_records():
    ax, ay, aw, ah, law, lah = _anchor_channels()
    half = np.float32(0.5)
    altx = ax - aw * half
    arbx = ax + aw * half
    alty = ay - ah * half
    arby = ay + ah * half
    chan = np.stack([ax, ay, aw, ah, law, lah], 0).reshape(6, NG, GSIZE)
    gminx = altx.reshape(NG, GSIZE).min(1)
    gmaxx = arbx.reshape(NG, GSIZE).max(1)
    gminy = alty.reshape(NG, GSIZE).min(1)
    gmaxy = arby.reshape(NG, GSIZE).max(1)
    npad_g = (A_PAD - A) // GSIZE
    gminx[NG - npad_g:] = np.float32(3e38)
    gmaxx[NG - npad_g:] = np.float32(-3e38)
    rec = np.zeros((NG, REC), np.float32)
    rec[:, 0] = gminx
    rec[:, 1] = gmaxx
    rec[:, 2] = gminy
    rec[:, 3] = gmaxy
    rec[:, 8:] = np.transpose(chan, (1, 0, 2)).reshape(NG, 6 * GSIZE)
    rec = rec.reshape(GPW, NW, REC).transpose(1, 0, 2)
    return np.ascontiguousarray(rec).reshape(-1)


_REC = _build_records()


_LOGC = (1.44251703, -0.71789838, 0.45689417,
         -0.27736505, 0.12191414, -0.0260663)
_LN2 = 0.6931471805599453


def _vlog(x):
    i = plsc.bitcast(x, jnp.int32)
    e = (i >> 23) - 127
    m = plsc.bitcast((i & 0x007FFFFF) | 0x3F800000, jnp.float32)
    t = m - 1.0
    p = jnp.float32(_LOGC[5])
    for k in range(4, -1, -1):
        p = jnp.float32(_LOGC[k]) + t * p
    p = t * p
    return (e.astype(jnp.float32) + p) * jnp.float32(_LN2)

_MESH = plsc.VectorSubcoreMesh(
    core_axis_name="c", subcore_axis_name="s", num_cores=NC, num_subcores=NS)


@functools.partial(
    pl.kernel,
    out_type=[
        jax.ShapeDtypeStruct((4 * A,), jnp.float32),
        jax.ShapeDtypeStruct((A,), jnp.int32),
        jax.ShapeDtypeStruct((A,), jnp.float32),
    ],
    mesh=_MESH,
    compiler_params=pltpu.CompilerParams(needs_layout_passes=False),
    scratch_types=[
        pltpu.VMEM((GPW * REC,), jnp.float32),
        pltpu.VMEM((NBV * L * 4,), jnp.float32),
        pltpu.VMEM((12 * TBOX,), jnp.float32),
        pltpu.VMEM((TBOX,), jnp.int32),
        pltpu.VMEM((TBOX,), jnp.int32),
        pltpu.VMEM((GPW * GSIZE,), jnp.float32),
        pltpu.VMEM((GPW * GSIZE,), jnp.float32),
        pltpu.VMEM((GPW * GSIZE,), jnp.float32),
        pltpu.VMEM((GPW * GSIZE,), jnp.float32),
        pltpu.VMEM((GPW * GSIZE,), jnp.int32),
        pltpu.VMEM((GPW * GSIZE,), jnp.float32),
        pltpu.SemaphoreType.DMA,
    ],
)
def _sc_encode(rec_h, box_h, lab_h,
               loc_hbm, cls_hbm, miou_hbm,
               rec_v, box_v, tab_v, lab_v, list_v,
               lx_v, ly_v, lw_v, lh_v, cls_v, miou_v, osem):
    w = lax.axis_index("s") * NC + lax.axis_index("c")
    pltpu.sync_copy(rec_h.at[pl.ds(w * (GPW * REC), GPW * REC)], rec_v)
    pltpu.sync_copy(box_h, box_v.at[pl.ds(0, NBOX * 4)])
    pltpu.sync_copy(lab_h, lab_v.at[pl.ds(0, NBOX)])

    zero = jnp.zeros((L,), jnp.float32)
    one = jnp.ones((L,), jnp.float32)
    zi = jnp.zeros((L,), jnp.int32)
    iota = lax.iota(jnp.int32, L)
    big = jnp.float32(3e38)

    for v in range(NBV):
        lane = iota + v * L
        idx = lane * 4
        x1 = plsc.load_gather(box_v, [idx])
        y1 = plsc.load_gather(box_v, [idx + 1])
        x2 = plsc.load_gather(box_v, [idx + 2])
        y2 = plsc.load_gather(box_v, [idx + 3])
        bx = (x1 + x2) * 0.5
        by = (y1 + y2) * 0.5
        bw = x2 - x1
        bh = y2 - y1
        hbw = bw * 0.5
        hbh = bh * 0.5
        valid = lane < NBOX
        sl = pl.ds(v * L, L)
        tab_v[sl] = jnp.where(valid, bx - hbw, big)
        tab_v[pl.ds(TBOX + v * L, L)] = jnp.where(valid, by - hbh, big)
        tab_v[pl.ds(2 * TBOX + v * L, L)] = jnp.where(valid, bx + hbw, -big)
        tab_v[pl.ds(3 * TBOX + v * L, L)] = jnp.where(valid, by + hbh, -big)
        tab_v[pl.ds(4 * TBOX + v * L, L)] = jnp.where(valid, bw * bh, zero)
        tab_v[pl.ds(5 * TBOX + v * L, L)] = bx
        tab_v[pl.ds(6 * TBOX + v * L, L)] = by
        tab_v[pl.ds(7 * TBOX + v * L, L)] = bw
        tab_v[pl.ds(8 * TBOX + v * L, L)] = bh
        tab_v[pl.ds(9 * TBOX + v * L, L)] = _vlog(jnp.where(valid, bw, one))
        tab_v[pl.ds(10 * TBOX + v * L, L)] = _vlog(jnp.where(valid, bh, one))

    def group_body(gl, carry):
        ro = gl * REC
        g = gl * NW + w
        rbase = lax.broadcast(ro, (L,))
        gminx = plsc.load_gather(rec_v, [rbase])
        gmaxx = plsc.load_gather(rec_v, [rbase + 1])
        gminy = plsc.load_gather(rec_v, [rbase + 2])
        gmaxy = plsc.load_gather(rec_v, [rbase + 3])

        off = zi
        for v in range(NBV):
            bltxv = tab_v[pl.ds(v * L, L)]
            bltyv = tab_v[pl.ds(TBOX + v * L, L)]
            brbxv = tab_v[pl.ds(2 * TBOX + v * L, L)]
            brbyv = tab_v[pl.ds(3 * TBOX + v * L, L)]
            incl = ((brbxv > gminx) & (bltxv < gmaxx)
                    & (brbyv > gminy) & (bltyv < gmaxy))
            pos = (off + plsc.cumsum(incl.astype(jnp.int32))) - 1
            plsc.store_scatter(list_v, [pos], iota + (v * L), mask=incl)
            off = off + plsc.all_reduce_population_count(incl)
        count = jnp.max(off)

        ab = ro + 8
        axs, ays, aws, ahs = [], [], [], []
        altx, alty, arbx, arby, area = [], [], [], [], []
        for k in range(UNROLL):
            axk = rec_v[pl.ds(ab + k * L, L)]
            ayk = rec_v[pl.ds(ab + GSIZE + k * L, L)]
            awk = rec_v[pl.ds(ab + 2 * GSIZE + k * L, L)]
            ahk = rec_v[pl.ds(ab + 3 * GSIZE + k * L, L)]
            hx = awk * 0.5
            hy = ahk * 0.5
            axs.append(axk)
            ays.append(ayk)
            aws.append(awk)
            ahs.append(ahk)
            altx.append(axk - hx)
            arbx.append(axk + hx)
            alty.append(ayk - hy)
            arby.append(ayk + hy)
            area.append(awk * ahk)

        plsc.store_scatter(list_v, [lax.broadcast(count, (L,))],
                           jnp.full((L,), NBOX, jnp.int32))

        def step(bc, cid):
            bi, bs, bid = bc
            bltx = plsc.load_gather(tab_v, [cid])
            blty = plsc.load_gather(tab_v, [cid + TBOX])
            brbx = plsc.load_gather(tab_v, [cid + 2 * TBOX])
            brby = plsc.load_gather(tab_v, [cid + 3 * TBOX])
            areab = plsc.load_gather(tab_v, [cid + 4 * TBOX])
            nbi, nbs, nbid = [], [], []
            for k in range(UNROLL):
                ltx = jnp.maximum(altx[k], bltx)
                lty = jnp.maximum(alty[k], blty)
                rbx = jnp.minimum(arbx[k], brbx)
                rby = jnp.minimum(arby[k], brby)
                wx = jnp.maximum(rbx - ltx, zero)
                wy = jnp.maximum(rby - lty, zero)
                inter = wx * wy
                sab = area[k] + areab
                p = inter * bs[k] > bi[k] * sab
                nbi.append(jnp.where(p, inter, bi[k]))
                nbs.append(jnp.where(p, sab, bs[k]))
                nbid.append(jnp.where(p, cid, bid[k]))
            return (tuple(nbi), tuple(nbs), tuple(nbid))

        def box_body(it, bc):
            jv = lax.broadcast(it * 2, (L,))
            cid0 = plsc.load_gather(list_v, [jv])
            cid1 = plsc.load_gather(list_v, [jv + 1])
            return step(step(bc, cid0), cid1)

        init = (tuple(zero for _ in range(UNROLL)),
                tuple(one for _ in range(UNROLL)),
                tuple(zi for _ in range(UNROLL)))
        binter, bsab, bid = lax.fori_loop(0, (count + 1) >> 1, box_body, init)

        for k in range(UNROLL):
            lo = gl * GSIZE + k * L
            idk = bid[k]
            bxg = plsc.load_gather(tab_v, [idk + 5 * TBOX])
            byg = plsc.load_gather(tab_v, [idk + 6 * TBOX])
            lbw = plsc.load_gather(tab_v, [idk + 9 * TBOX])
            lbh = plsc.load_gather(tab_v, [idk + 10 * TBOX])
            areab = plsc.load_gather(tab_v, [idk + 4 * TBOX])
            lab = plsc.load_gather(lab_v, [idk])
            lx = (bxg - axs[k]) / aws[k]
            ly = (byg - ays[k]) / ahs[k]
            lw = lbw - rec_v[pl.ds(ab + 4 * GSIZE + k * L, L)]
            lh = lbh - rec_v[pl.ds(ab + 5 * GSIZE + k * L, L)]
            denom = (area[k] + areab) - binter[k]
            miou = binter[k] / denom
            cls = jnp.where(miou > 0.5, lab + 1,
                            jnp.where(miou > 0.4, -1, 0))
            sl = pl.ds(lo, L)
            lx_v[sl] = lx
            ly_v[sl] = ly
            lw_v[sl] = lw
            lh_v[sl] = lh
            cls_v[sl] = cls
            miou_v[sl] = miou

        @pl.when(g < NR)
        def _issue_out():
            src = pl.ds(gl * GSIZE, GSIZE)
            dst = g * GSIZE
            pltpu.async_copy(lx_v.at[src], loc_hbm.at[pl.ds(dst, GSIZE)], osem)
            pltpu.async_copy(ly_v.at[src], loc_hbm.at[pl.ds(A + dst, GSIZE)], osem)
            pltpu.async_copy(lw_v.at[src], loc_hbm.at[pl.ds(2 * A + dst, GSIZE)], osem)
            pltpu.async_copy(lh_v.at[src], loc_hbm.at[pl.ds(3 * A + dst, GSIZE)], osem)
            pltpu.async_copy(cls_v.at[src], cls_hbm.at[pl.ds(dst, GSIZE)], osem)
            pltpu.async_copy(miou_v.at[src], miou_hbm.at[pl.ds(dst, GSIZE)], osem)

        @pl.when(gl >= DEPTH)
        def _drain_one():
            pltpu.make_async_copy(
                loc_hbm.at[pl.ds(0, 6 * GSIZE)],
                lx_v.at[pl.ds(0, 6 * GSIZE)], osem).wait()

        return carry

    lax.fori_loop(0, GPW, group_body, 0)

    @pl.when(w < NR - (GPW - 1) * NW)
    def _drain_full():
        pltpu.make_async_copy(
            loc_hbm.at[pl.ds(0, DEPTH * 6 * GSIZE)],
            lx_v.at[pl.ds(0, DEPTH * 6 * GSIZE)], osem).wait()

    @pl.when(w >= NR - (GPW - 1) * NW)
    def _drain_short():
        pltpu.make_async_copy(
            loc_hbm.at[pl.ds(0, (DEPTH - 1) * 6 * GSIZE)],
            lx_v.at[pl.ds(0, (DEPTH - 1) * 6 * GSIZE)], osem).wait()


def kernel(boxes, labels, input_size):
    del input_size
    boxes = boxes.astype(jnp.float32).reshape(-1)
    labels = labels.astype(jnp.int32)
    rec = jnp.asarray(_REC)
    loc_planes, cls, miou = _sc_encode(rec, boxes, labels)
    loc = loc_planes.reshape(4, A).T
    return loc, cls, miou


# speedup vs baseline: 1.1900x; 1.0628x over previous
import functools
import math

import jax
import jax.numpy as jnp
import numpy as np
from jax import lax
from jax.experimental import pallas as pl
from jax.experimental.pallas import tpu as pltpu
from jax.experimental.pallas import tpu_sc as plsc

ANCHOR_AREAS = [1024.0, 4096.0, 16384.0, 65536.0, 262144.0]
ASPECT_RATIOS = [0.5, 1.0, 2.0]
SCALE_RATIOS = [1.0, 2.0 ** (1.0 / 3.0), 2.0 ** (2.0 / 3.0)]
INPUT_W, INPUT_H = 1024.0, 1024.0

NC, NS, L = 2, 16, 16
NW = NC * NS
A = 196416
A_PAD = 196608
GSIZE = 96
NG = A_PAD // GSIZE
NR = A // GSIZE
GPW = NG // NW
UNROLL = GSIZE // L
NBOX = 200
NBV = 13
TBOX = 256
REC = 8 + 6 * GSIZE
KPP = UNROLL // 2
DEPTH = 8


def _anchor_channels():
    wh = []
    for s in ANCHOR_AREAS:
        for ar in ASPECT_RATIOS:
            h = math.sqrt(s / ar)
            w = ar * h
            for sr in SCALE_RATIOS:
                wh.append([w * sr, h * sr])
    num_fms = len(ANCHOR_AREAS)
    num_anchors = len(ASPECT_RATIOS) * len(SCALE_RATIOS)
    anchor_wh = np.asarray(wh, dtype=np.float32).reshape(num_fms, -1, 2)
    boxes = []
    for i in range(num_fms):
        fm_w = int(math.ceil(INPUT_W / 2.0 ** (i + 3)))
        fm_h = int(math.ceil(INPUT_H / 2.0 ** (i + 3)))
        gx = INPUT_W / fm_w
        gy = INPUT_H / fm_h
        xs = (np.arange(fm_w, dtype=np.float32) + 0.5) * gx
        ys = (np.arange(fm_h, dtype=np.float32) + 0.5) * gy
        xv, yv = np.meshgrid(xs, ys)
        xy = np.stack([xv, yv], axis=-1).reshape(fm_h, fm_w, 1, 2)
        xy = np.broadcast_to(xy, (fm_h, fm_w, num_anchors, 2))
        whb = np.broadcast_to(
            anchor_wh[i].reshape(1, 1, num_anchors, 2), (fm_h, fm_w, num_anchors, 2))
        box = np.concatenate([xy, whb], axis=3)
        boxes.append(box.reshape(-1, 4))
    anc = np.concatenate(boxes, axis=0).astype(np.float32)
    pad = np.zeros((A_PAD - A, 4), np.float32)
    pad[:, 2:] = 1.0
    anc = np.concatenate([anc, pad], axis=0)
    ax, ay, aw, ah = (anc[:, j].copy() for j in range(4))
    return (ax, ay, aw, ah,
            np.log(aw).astype(np.float32), np.log(ah).astype(np.float32))


def _build_records():
    ax, ay, aw, ah, law, lah = _anchor_channels()
    half = np.float32(0.5)
    altx = ax - aw * half
    arbx = ax + aw * half
    alty = ay - ah * half
    arby = ay + ah * half
    chan = np.stack([ax, ay, aw, ah, law, lah], 0).reshape(6, NG, GSIZE)
    gminx = altx.reshape(NG, GSIZE).min(1)
    gmaxx = arbx.reshape(NG, GSIZE).max(1)
    gminy = alty.reshape(NG, GSIZE).min(1)
    gmaxy = arby.reshape(NG, GSIZE).max(1)
    npad_g = (A_PAD - A) // GSIZE
    gminx[NG - npad_g:] = np.float32(3e38)
    gmaxx[NG - npad_g:] = np.float32(-3e38)
    rec = np.zeros((NG, REC), np.float32)
    rec[:, 0] = gminx
    rec[:, 1] = gmaxx
    rec[:, 2] = gminy
    rec[:, 3] = gmaxy
    rec[:, 8:] = np.transpose(chan, (1, 0, 2)).reshape(NG, 6 * GSIZE)
    rec = rec.reshape(GPW, NW, REC).transpose(1, 0, 2)
    return np.ascontiguousarray(rec).reshape(-1)


_REC = _build_records()


_LOGC = (1.44251703, -0.71789838, 0.45689417,
         -0.27736505, 0.12191414, -0.0260663)
_LN2 = 0.6931471805599453


def _vlog(x):
    i = plsc.bitcast(x, jnp.int32)
    e = (i >> 23) - 127
    m = plsc.bitcast((i & 0x007FFFFF) | 0x3F800000, jnp.float32)
    t = m - 1.0
    p = jnp.float32(_LOGC[5])
    for k in range(4, -1, -1):
        p = jnp.float32(_LOGC[k]) + t * p
    p = t * p
    return (e.astype(jnp.float32) + p) * jnp.float32(_LN2)

_MESH = plsc.VectorSubcoreMesh(
    core_axis_name="c", subcore_axis_name="s", num_cores=NC, num_subcores=NS)


@functools.partial(
    pl.kernel,
    out_type=[
        jax.ShapeDtypeStruct((4 * A,), jnp.float32),
        jax.ShapeDtypeStruct((A,), jnp.int32),
        jax.ShapeDtypeStruct((A,), jnp.float32),
    ],
    mesh=_MESH,
    compiler_params=pltpu.CompilerParams(needs_layout_passes=False),
    scratch_types=[
        pltpu.VMEM((GPW * REC,), jnp.float32),
        pltpu.VMEM((NBV * L * 4,), jnp.float32),
        pltpu.VMEM((12 * TBOX,), jnp.float32),
        pltpu.VMEM((TBOX,), jnp.int32),
        pltpu.VMEM((TBOX,), jnp.int32),
        pltpu.VMEM((GPW * GSIZE,), jnp.float32),
        pltpu.VMEM((GPW * GSIZE,), jnp.float32),
        pltpu.VMEM((GPW * GSIZE,), jnp.float32),
        pltpu.VMEM((GPW * GSIZE,), jnp.float32),
        pltpu.VMEM((GPW * GSIZE,), jnp.int32),
        pltpu.VMEM((GPW * GSIZE,), jnp.float32),
        pltpu.SemaphoreType.DMA,
    ],
)
def _sc_encode(rec_h, box_h, lab_h,
               loc_hbm, cls_hbm, miou_hbm,
               rec_v, box_v, tab_v, lab_v, list_v,
               lx_v, ly_v, lw_v, lh_v, cls_v, miou_v, osem):
    w = lax.axis_index("s") * NC + lax.axis_index("c")
    pltpu.sync_copy(rec_h.at[pl.ds(w * (GPW * REC), GPW * REC)], rec_v)
    pltpu.sync_copy(box_h, box_v.at[pl.ds(0, NBOX * 4)])
    pltpu.sync_copy(lab_h, lab_v.at[pl.ds(0, NBOX)])

    zero = jnp.zeros((L,), jnp.float32)
    one = jnp.ones((L,), jnp.float32)
    zi = jnp.zeros((L,), jnp.int32)
    iota = lax.iota(jnp.int32, L)
    big = jnp.float32(3e38)

    for v in range(NBV):
        lane = iota + v * L
        idx = lane * 4
        x1 = plsc.load_gather(box_v, [idx])
        y1 = plsc.load_gather(box_v, [idx + 1])
        x2 = plsc.load_gather(box_v, [idx + 2])
        y2 = plsc.load_gather(box_v, [idx + 3])
        bx = (x1 + x2) * 0.5
        by = (y1 + y2) * 0.5
        bw = x2 - x1
        bh = y2 - y1
        hbw = bw * 0.5
        hbh = bh * 0.5
        valid = lane < NBOX
        sl = pl.ds(v * L, L)
        tab_v[sl] = jnp.where(valid, bx - hbw, big)
        tab_v[pl.ds(TBOX + v * L, L)] = jnp.where(valid, by - hbh, big)
        tab_v[pl.ds(2 * TBOX + v * L, L)] = jnp.where(valid, bx + hbw, -big)
        tab_v[pl.ds(3 * TBOX + v * L, L)] = jnp.where(valid, by + hbh, -big)
        tab_v[pl.ds(4 * TBOX + v * L, L)] = jnp.where(valid, bw * bh, zero)
        tab_v[pl.ds(5 * TBOX + v * L, L)] = bx
        tab_v[pl.ds(6 * TBOX + v * L, L)] = by
        tab_v[pl.ds(7 * TBOX + v * L, L)] = bw
        tab_v[pl.ds(8 * TBOX + v * L, L)] = bh
        tab_v[pl.ds(9 * TBOX + v * L, L)] = _vlog(jnp.where(valid, bw, one))
        tab_v[pl.ds(10 * TBOX + v * L, L)] = _vlog(jnp.where(valid, bh, one))

    def group_body(gl, carry):
        ro = gl * REC
        g = gl * NW + w
        rbase = lax.broadcast(ro, (L,))
        gminx = plsc.load_gather(rec_v, [rbase])
        gmaxx = plsc.load_gather(rec_v, [rbase + 1])
        gminy = plsc.load_gather(rec_v, [rbase + 2])
        gmaxy = plsc.load_gather(rec_v, [rbase + 3])

        off = zi
        for v in range(NBV):
            bltxv = tab_v[pl.ds(v * L, L)]
            bltyv = tab_v[pl.ds(TBOX + v * L, L)]
            brbxv = tab_v[pl.ds(2 * TBOX + v * L, L)]
            brbyv = tab_v[pl.ds(3 * TBOX + v * L, L)]
            incl = ((brbxv > gminx) & (bltxv < gmaxx)
                    & (brbyv > gminy) & (bltyv < gmaxy))
            pos = (off + plsc.cumsum(incl.astype(jnp.int32))) - 1
            plsc.store_scatter(list_v, [pos], iota + (v * L), mask=incl)
            off = off + plsc.all_reduce_population_count(incl)
        count = jnp.max(off)

        ab = ro + 8
        axs, ays, aws, ahs = [], [], [], []
        altx, alty, arbx, arby, area = [], [], [], [], []
        for k in range(UNROLL):
            axk = rec_v[pl.ds(ab + k * L, L)]
            ayk = rec_v[pl.ds(ab + GSIZE + k * L, L)]
            awk = rec_v[pl.ds(ab + 2 * GSIZE + k * L, L)]
            ahk = rec_v[pl.ds(ab + 3 * GSIZE + k * L, L)]
            hx = awk * 0.5
            hy = ahk * 0.5
            axs.append(axk)
            ays.append(ayk)
            aws.append(awk)
            ahs.append(ahk)
            altx.append(axk - hx)
            arbx.append(axk + hx)
            alty.append(ayk - hy)
            arby.append(ayk + hy)
            area.append(awk * ahk)

        def run_pass(k0):
            def box_body(it, bc):
                bi, bs, bid = bc
                cid = plsc.load_gather(list_v, [lax.broadcast(it, (L,))])
                bltx = plsc.load_gather(tab_v, [cid])
                blty = plsc.load_gather(tab_v, [cid + TBOX])
                brbx = plsc.load_gather(tab_v, [cid + 2 * TBOX])
                brby = plsc.load_gather(tab_v, [cid + 3 * TBOX])
                areab = plsc.load_gather(tab_v, [cid + 4 * TBOX])
                nbi, nbs, nbid = [], [], []
                for kk in range(KPP):
                    k = k0 + kk
                    ltx = jnp.maximum(altx[k], bltx)
                    lty = jnp.maximum(alty[k], blty)
                    rbx = jnp.minimum(arbx[k], brbx)
                    rby = jnp.minimum(arby[k], brby)
                    wx = jnp.maximum(rbx - ltx, zero)
                    wy = jnp.maximum(rby - lty, zero)
                    inter = wx * wy
                    sab = area[k] + areab
                    p = inter * bs[kk] > bi[kk] * sab
                    nbi.append(jnp.where(p, inter, bi[kk]))
                    nbs.append(jnp.where(p, sab, bs[kk]))
                    nbid.append(jnp.where(p, cid, bid[kk]))
                return (tuple(nbi), tuple(nbs), tuple(nbid))

            init = (tuple(zero for _ in range(KPP)),
                    tuple(one for _ in range(KPP)),
                    tuple(zi for _ in range(KPP)))
            return lax.fori_loop(0, count, box_body, init)

        bi0, bs0, bid0 = run_pass(0)
        bi1, bs1, bid1 = run_pass(KPP)
        binter = bi0 + bi1
        bsab = bs0 + bs1
        bid = bid0 + bid1

        for k in range(UNROLL):
            lo = gl * GSIZE + k * L
            idk = bid[k]
            bxg = plsc.load_gather(tab_v, [idk + 5 * TBOX])
            byg = plsc.load_gather(tab_v, [idk + 6 * TBOX])
            lbw = plsc.load_gather(tab_v, [idk + 9 * TBOX])
            lbh = plsc.load_gather(tab_v, [idk + 10 * TBOX])
            areab = plsc.load_gather(tab_v, [idk + 4 * TBOX])
            lab = plsc.load_gather(lab_v, [idk])
            lx = (bxg - axs[k]) / aws[k]
            ly = (byg - ays[k]) / ahs[k]
            lw = lbw - rec_v[pl.ds(ab + 4 * GSIZE + k * L, L)]
            lh = lbh - rec_v[pl.ds(ab + 5 * GSIZE + k * L, L)]
            denom = (area[k] + areab) - binter[k]
            miou = binter[k] / denom
            cls = jnp.where(miou > 0.5, lab + 1,
                            jnp.where(miou > 0.4, -1, 0))
            sl = pl.ds(lo, L)
            lx_v[sl] = lx
            ly_v[sl] = ly
            lw_v[sl] = lw
            lh_v[sl] = lh
            cls_v[sl] = cls
            miou_v[sl] = miou

        @pl.when(g < NR)
        def _issue_out():
            src = pl.ds(gl * GSIZE, GSIZE)
            dst = g * GSIZE
            pltpu.async_copy(lx_v.at[src], loc_hbm.at[pl.ds(dst, GSIZE)], osem)
            pltpu.async_copy(ly_v.at[src], loc_hbm.at[pl.ds(A + dst, GSIZE)], osem)
            pltpu.async_copy(lw_v.at[src], loc_hbm.at[pl.ds(2 * A + dst, GSIZE)], osem)
            pltpu.async_copy(lh_v.at[src], loc_hbm.at[pl.ds(3 * A + dst, GSIZE)], osem)
            pltpu.async_copy(cls_v.at[src], cls_hbm.at[pl.ds(dst, GSIZE)], osem)
            pltpu.async_copy(miou_v.at[src], miou_hbm.at[pl.ds(dst, GSIZE)], osem)

        @pl.when(gl >= DEPTH)
        def _drain_one():
            pltpu.make_async_copy(
                loc_hbm.at[pl.ds(0, 6 * GSIZE)],
                lx_v.at[pl.ds(0, 6 * GSIZE)], osem).wait()

        return carry

    lax.fori_loop(0, GPW, group_body, 0)

    @pl.when(w < NR - (GPW - 1) * NW)
    def _drain_full():
        pltpu.make_async_copy(
            loc_hbm.at[pl.ds(0, DEPTH * 6 * GSIZE)],
            lx_v.at[pl.ds(0, DEPTH * 6 * GSIZE)], osem).wait()

    @pl.when(w >= NR - (GPW - 1) * NW)
    def _drain_short():
        pltpu.make_async_copy(
            loc_hbm.at[pl.ds(0, (DEPTH - 1) * 6 * GSIZE)],
            lx_v.at[pl.ds(0, (DEPTH - 1) * 6 * GSIZE)], osem).wait()


def kernel(boxes, labels, input_size):
    del input_size
    boxes = boxes.astype(jnp.float32).reshape(-1)
    labels = labels.astype(jnp.int32)
    rec = jnp.asarray(_REC)
    loc_planes, cls, miou = _sc_encode(rec, boxes, labels)
    loc = loc_planes.reshape(4, A).T
    return loc, cls, miou


# speedup vs baseline: 1.2010x; 1.0092x over previous
import functools
import math

import jax
import jax.numpy as jnp
import numpy as np
from jax import lax
from jax.experimental import pallas as pl
from jax.experimental.pallas import tpu as pltpu
from jax.experimental.pallas import tpu_sc as plsc

ANCHOR_AREAS = [1024.0, 4096.0, 16384.0, 65536.0, 262144.0]
ASPECT_RATIOS = [0.5, 1.0, 2.0]
SCALE_RATIOS = [1.0, 2.0 ** (1.0 / 3.0), 2.0 ** (2.0 / 3.0)]
INPUT_W, INPUT_H = 1024.0, 1024.0

NC, NS, L = 2, 16, 16
NW = NC * NS
A = 196416
A_PAD = 196608
GSIZE = 96
NG = A_PAD // GSIZE
NR = A // GSIZE
GPW = NG // NW
UNROLL = GSIZE // L
NBOX = 200
NBV = 13
TBOX = 256
REC = 8 + 6 * GSIZE
KPP = UNROLL // 2
DEPTH = 8


def _anchor_channels():
    wh = []
    for s in ANCHOR_AREAS:
        for ar in ASPECT_RATIOS:
            h = math.sqrt(s / ar)
            w = ar * h
            for sr in SCALE_RATIOS:
                wh.append([w * sr, h * sr])
    num_fms = len(ANCHOR_AREAS)
    num_anchors = len(ASPECT_RATIOS) * len(SCALE_RATIOS)
    anchor_wh = np.asarray(wh, dtype=np.float32).reshape(num_fms, -1, 2)
    boxes = []
    for i in range(num_fms):
        fm_w = int(math.ceil(INPUT_W / 2.0 ** (i + 3)))
        fm_h = int(math.ceil(INPUT_H / 2.0 ** (i + 3)))
        gx = INPUT_W / fm_w
        gy = INPUT_H / fm_h
        xs = (np.arange(fm_w, dtype=np.float32) + 0.5) * gx
        ys = (np.arange(fm_h, dtype=np.float32) + 0.5) * gy
        xv, yv = np.meshgrid(xs, ys)
        xy = np.stack([xv, yv], axis=-1).reshape(fm_h, fm_w, 1, 2)
        xy = np.broadcast_to(xy, (fm_h, fm_w, num_anchors, 2))
        whb = np.broadcast_to(
            anchor_wh[i].reshape(1, 1, num_anchors, 2), (fm_h, fm_w, num_anchors, 2))
        box = np.concatenate([xy, whb], axis=3)
        boxes.append(box.reshape(-1, 4))
    anc = np.concatenate(boxes, axis=0).astype(np.float32)
    pad = np.zeros((A_PAD - A, 4), np.float32)
    pad[:, 2:] = 1.0
    anc = np.concatenate([anc, pad], axis=0)
    ax, ay, aw, ah = (anc[:, j].copy() for j in range(4))
    return (ax, ay, aw, ah,
            np.log(aw).astype(np.float32), np.log(ah).astype(np.float32))


def _build_records():
    ax, ay, aw, ah, law, lah = _anchor_channels()
    half = np.float32(0.5)
    altx = ax - aw * half
    arbx = ax + aw * half
    alty = ay - ah * half
    arby = ay + ah * half
    chan = np.stack([ax, ay, aw, ah, law, lah], 0).reshape(6, NG, GSIZE)
    gminx = altx.reshape(NG, GSIZE).min(1)
    gmaxx = arbx.reshape(NG, GSIZE).max(1)
    gminy = alty.reshape(NG, GSIZE).min(1)
    gmaxy = arby.reshape(NG, GSIZE).max(1)
    npad_g = (A_PAD - A) // GSIZE
    gminx[NG - npad_g:] = np.float32(3e38)
    gmaxx[NG - npad_g:] = np.float32(-3e38)
    rec = np.zeros((NG, REC), np.float32)
    rec[:, 0] = gminx
    rec[:, 1] = gmaxx
    rec[:, 2] = gminy
    rec[:, 3] = gmaxy
    rec[:, 8:] = np.transpose(chan, (1, 0, 2)).reshape(NG, 6 * GSIZE)
    rec = rec.reshape(GPW, NW, REC).transpose(1, 0, 2)
    return np.ascontiguousarray(rec).reshape(-1)


_REC = _build_records()


_LOGC = (1.44251703, -0.71789838, 0.45689417,
         -0.27736505, 0.12191414, -0.0260663)
_LN2 = 0.6931471805599453


def _vlog(x):
    i = plsc.bitcast(x, jnp.int32)
    e = (i >> 23) - 127
    m = plsc.bitcast((i & 0x007FFFFF) | 0x3F800000, jnp.float32)
    t = m - 1.0
    p = jnp.float32(_LOGC[5])
    for k in range(4, -1, -1):
        p = jnp.float32(_LOGC[k]) + t * p
    p = t * p
    return (e.astype(jnp.float32) + p) * jnp.float32(_LN2)

_MESH = plsc.VectorSubcoreMesh(
    core_axis_name="c", subcore_axis_name="s", num_cores=NC, num_subcores=NS)


@functools.partial(
    pl.kernel,
    out_type=[
        jax.ShapeDtypeStruct((4 * A,), jnp.float32),
        jax.ShapeDtypeStruct((A,), jnp.int32),
        jax.ShapeDtypeStruct((A,), jnp.float32),
    ],
    mesh=_MESH,
    compiler_params=pltpu.CompilerParams(needs_layout_passes=False),
    scratch_types=[
        pltpu.VMEM((GPW * REC,), jnp.float32),
        pltpu.VMEM((NBV * L * 4,), jnp.float32),
        pltpu.VMEM((12 * TBOX,), jnp.float32),
        pltpu.VMEM((TBOX,), jnp.int32),
        pltpu.VMEM((TBOX,), jnp.int32),
        pltpu.VMEM((GPW * GSIZE,), jnp.float32),
        pltpu.VMEM((GPW * GSIZE,), jnp.float32),
        pltpu.VMEM((GPW * GSIZE,), jnp.float32),
        pltpu.VMEM((GPW * GSIZE,), jnp.float32),
        pltpu.VMEM((GPW * GSIZE,), jnp.int32),
        pltpu.VMEM((GPW * GSIZE,), jnp.float32),
        pltpu.SemaphoreType.DMA,
        pltpu.SemaphoreType.DMA,
    ],
)
def _sc_encode(rec_h, box_h, lab_h,
               loc_hbm, cls_hbm, miou_hbm,
               rec_v, box_v, tab_v, lab_v, list_v,
               lx_v, ly_v, lw_v, lh_v, cls_v, miou_v, osem, rsem):
    w = lax.axis_index("s") * NC + lax.axis_index("c")
    rec_dma = pltpu.async_copy(
        rec_h.at[pl.ds(w * (GPW * REC), GPW * REC)], rec_v, rsem)
    pltpu.sync_copy(box_h, box_v.at[pl.ds(0, NBOX * 4)])
    pltpu.sync_copy(lab_h, lab_v.at[pl.ds(0, NBOX)])

    zero = jnp.zeros((L,), jnp.float32)
    one = jnp.ones((L,), jnp.float32)
    zi = jnp.zeros((L,), jnp.int32)
    iota = lax.iota(jnp.int32, L)
    big = jnp.float32(3e38)

    for v in range(NBV):
        lane = iota + v * L
        idx = lane * 4
        x1 = plsc.load_gather(box_v, [idx])
        y1 = plsc.load_gather(box_v, [idx + 1])
        x2 = plsc.load_gather(box_v, [idx + 2])
        y2 = plsc.load_gather(box_v, [idx + 3])
        bx = (x1 + x2) * 0.5
        by = (y1 + y2) * 0.5
        bw = x2 - x1
        bh = y2 - y1
        hbw = bw * 0.5
        hbh = bh * 0.5
        valid = lane < NBOX
        sl = pl.ds(v * L, L)
        tab_v[sl] = jnp.where(valid, bx - hbw, big)
        tab_v[pl.ds(TBOX + v * L, L)] = jnp.where(valid, by - hbh, big)
        tab_v[pl.ds(2 * TBOX + v * L, L)] = jnp.where(valid, bx + hbw, -big)
        tab_v[pl.ds(3 * TBOX + v * L, L)] = jnp.where(valid, by + hbh, -big)
        tab_v[pl.ds(4 * TBOX + v * L, L)] = jnp.where(valid, bw * bh, zero)
        tab_v[pl.ds(5 * TBOX + v * L, L)] = bx
        tab_v[pl.ds(6 * TBOX + v * L, L)] = by
        tab_v[pl.ds(7 * TBOX + v * L, L)] = bw
        tab_v[pl.ds(8 * TBOX + v * L, L)] = bh
        tab_v[pl.ds(9 * TBOX + v * L, L)] = _vlog(jnp.where(valid, bw, one))
        tab_v[pl.ds(10 * TBOX + v * L, L)] = _vlog(jnp.where(valid, bh, one))

    rec_dma.wait()

    def group_body(gl, carry):
        ro = gl * REC
        g = gl * NW + w
        rbase = lax.broadcast(ro, (L,))
        gminx = plsc.load_gather(rec_v, [rbase])
        gmaxx = plsc.load_gather(rec_v, [rbase + 1])
        gminy = plsc.load_gather(rec_v, [rbase + 2])
        gmaxy = plsc.load_gather(rec_v, [rbase + 3])

        off = zi
        for v in range(NBV):
            bltxv = tab_v[pl.ds(v * L, L)]
            bltyv = tab_v[pl.ds(TBOX + v * L, L)]
            brbxv = tab_v[pl.ds(2 * TBOX + v * L, L)]
            brbyv = tab_v[pl.ds(3 * TBOX + v * L, L)]
            incl = ((brbxv > gminx) & (bltxv < gmaxx)
                    & (brbyv > gminy) & (bltyv < gmaxy))
            pos = (off + plsc.cumsum(incl.astype(jnp.int32))) - 1
            plsc.store_scatter(list_v, [pos], iota + (v * L), mask=incl)
            off = off + plsc.all_reduce_population_count(incl)
        count = jnp.max(off)

        ab = ro + 8
        axs, ays, aws, ahs = [], [], [], []
        altx, alty, arbx, arby, area = [], [], [], [], []
        for k in range(UNROLL):
            axk = rec_v[pl.ds(ab + k * L, L)]
            ayk = rec_v[pl.ds(ab + GSIZE + k * L, L)]
            awk = rec_v[pl.ds(ab + 2 * GSIZE + k * L, L)]
            ahk = rec_v[pl.ds(ab + 3 * GSIZE + k * L, L)]
            hx = awk * 0.5
            hy = ahk * 0.5
            axs.append(axk)
            ays.append(ayk)
            aws.append(awk)
            ahs.append(ahk)
            altx.append(axk - hx)
            arbx.append(axk + hx)
            alty.append(ayk - hy)
            arby.append(ayk + hy)
            area.append(awk * ahk)

        def run_pass(k0):
            def box_body(it, bc):
                bi, bs, bid = bc
                cid = plsc.load_gather(list_v, [lax.broadcast(it, (L,))])
                bltx = plsc.load_gather(tab_v, [cid])
                blty = plsc.load_gather(tab_v, [cid + TBOX])
                brbx = plsc.load_gather(tab_v, [cid + 2 * TBOX])
                brby = plsc.load_gather(tab_v, [cid + 3 * TBOX])
                areab = plsc.load_gather(tab_v, [cid + 4 * TBOX])
                nbi, nbs, nbid = [], [], []
                for kk in range(KPP):
                    k = k0 + kk
                    ltx = jnp.maximum(altx[k], bltx)
                    lty = jnp.maximum(alty[k], blty)
                    rbx = jnp.minimum(arbx[k], brbx)
                    rby = jnp.minimum(arby[k], brby)
                    wx = jnp.maximum(rbx - ltx, zero)
                    wy = jnp.maximum(rby - lty, zero)
                    inter = wx * wy
                    sab = area[k] + areab
                    p = inter * bs[kk] > bi[kk] * sab
                    nbi.append(jnp.where(p, inter, bi[kk]))
                    nbs.append(jnp.where(p, sab, bs[kk]))
                    nbid.append(jnp.where(p, cid, bid[kk]))
                return (tuple(nbi), tuple(nbs), tuple(nbid))

            init = (tuple(zero for _ in range(KPP)),
                    tuple(one for _ in range(KPP)),
                    tuple(zi for _ in range(KPP)))
            return lax.fori_loop(0, count, box_body, init)

        bi0, bs0, bid0 = run_pass(0)
        bi1, bs1, bid1 = run_pass(KPP)
        binter = bi0 + bi1
        bsab = bs0 + bs1
        bid = bid0 + bid1

        for k in range(UNROLL):
            lo = gl * GSIZE + k * L
            idk = bid[k]
            bxg = plsc.load_gather(tab_v, [idk + 5 * TBOX])
            byg = plsc.load_gather(tab_v, [idk + 6 * TBOX])
            lbw = plsc.load_gather(tab_v, [idk + 9 * TBOX])
            lbh = plsc.load_gather(tab_v, [idk + 10 * TBOX])
            areab = plsc.load_gather(tab_v, [idk + 4 * TBOX])
            lab = plsc.load_gather(lab_v, [idk])
            lx = (bxg - axs[k]) / aws[k]
            ly = (byg - ays[k]) / ahs[k]
            lw = lbw - rec_v[pl.ds(ab + 4 * GSIZE + k * L, L)]
            lh = lbh - rec_v[pl.ds(ab + 5 * GSIZE + k * L, L)]
            denom = (area[k] + areab) - binter[k]
            miou = binter[k] / denom
            cls = jnp.where(miou > 0.5, lab + 1,
                            jnp.where(miou > 0.4, -1, 0))
            sl = pl.ds(lo, L)
            lx_v[sl] = lx
            ly_v[sl] = ly
            lw_v[sl] = lw
            lh_v[sl] = lh
            cls_v[sl] = cls
            miou_v[sl] = miou

        @pl.when(g < NR)
        def _issue_out():
            src = pl.ds(gl * GSIZE, GSIZE)
            dst = g * GSIZE
            pltpu.async_copy(lx_v.at[src], loc_hbm.at[pl.ds(dst, GSIZE)], osem)
            pltpu.async_copy(ly_v.at[src], loc_hbm.at[pl.ds(A + dst, GSIZE)], osem)
            pltpu.async_copy(lw_v.at[src], loc_hbm.at[pl.ds(2 * A + dst, GSIZE)], osem)
            pltpu.async_copy(lh_v.at[src], loc_hbm.at[pl.ds(3 * A + dst, GSIZE)], osem)
            pltpu.async_copy(cls_v.at[src], cls_hbm.at[pl.ds(dst, GSIZE)], osem)
            pltpu.async_copy(miou_v.at[src], miou_hbm.at[pl.ds(dst, GSIZE)], osem)

        @pl.when(gl >= DEPTH)
        def _drain_one():
            pltpu.make_async_copy(
                loc_hbm.at[pl.ds(0, 6 * GSIZE)],
                lx_v.at[pl.ds(0, 6 * GSIZE)], osem).wait()

        return carry

    lax.fori_loop(0, GPW, group_body, 0)

    @pl.when(w < NR - (GPW - 1) * NW)
    def _drain_full():
        pltpu.make_async_copy(
            loc_hbm.at[pl.ds(0, DEPTH * 6 * GSIZE)],
            lx_v.at[pl.ds(0, DEPTH * 6 * GSIZE)], osem).wait()

    @pl.when(w >= NR - (GPW - 1) * NW)
    def _drain_short():
        pltpu.make_async_copy(
            loc_hbm.at[pl.ds(0, (DEPTH - 1) * 6 * GSIZE)],
            lx_v.at[pl.ds(0, (DEPTH - 1) * 6 * GSIZE)], osem).wait()


def kernel(boxes, labels, input_size):
    del input_size
    boxes = boxes.astype(jnp.float32).reshape(-1)
    labels = labels.astype(jnp.int32)
    rec = jnp.asarray(_REC)
    loc_planes, cls, miou = _sc_encode(rec, boxes, labels)
    loc = loc_planes.reshape(4, A).T
    return loc, cls, miou
